# Initial kernel scaffold; baseline (speedup 1.0000x reference)
import jax
import jax.numpy as jnp
from jax.experimental import pallas as pl


def kernel(x, embed, W1, b1, W2, b2):
    raise NotImplementedError("write your pallas kernel here")


# speedup vs baseline: 9.8841x; 9.8841x over previous
import functools

import jax
import jax.numpy as jnp
from jax import lax
from jax.experimental import pallas as pl
from jax.experimental.pallas import tpu as pltpu
from jax.experimental.pallas import tpu_sc as plsc

NUM_CORES = 2
NUM_SUBCORES = 16
NUM_WORKERS = NUM_CORES * NUM_SUBCORES
LANES = 16


@functools.cache
def _make_sc_pool(B, S, D, V):
    assert B % NUM_WORKERS == 0
    bw = B // NUM_WORKERS
    assert bw % 8 == 0 and bw <= 128
    assert D % LANES == 0
    nc = D // LANES
    assert S % 2 == 0

    mesh = plsc.VectorSubcoreMesh(core_axis_name="c", subcore_axis_name="s")

    @functools.partial(
        pl.kernel,
        mesh=mesh,
        out_type=jax.ShapeDtypeStruct((B, D), jnp.float32),
        scratch_types=[
            pltpu.VMEM((S, bw), jnp.int32),
            pltpu.VMEM((bw, D), jnp.float32),
            pltpu.VMEM((bw, D), jnp.float32),
            pltpu.VMEM((bw, D), jnp.float32),
            pltpu.SemaphoreType.DMA,
            pltpu.SemaphoreType.DMA,
        ],
    )
    def sc_pool(xT_hbm, table_hbm, out_hbm, idx_v, acc_v, buf0, buf1, sem0, sem1):
        wid = lax.axis_index("s") * NUM_CORES + lax.axis_index("c")
        base = wid * bw

        pltpu.sync_copy(xT_hbm.at[:, pl.ds(base, bw)], idx_v)

        zeros = jnp.zeros((LANES,), jnp.float32)

        def zero_row(r, carry):
            for c in range(nc):
                acc_v[r, pl.ds(c * LANES, LANES)] = zeros
            return carry

        lax.fori_loop(0, bw, zero_row, 0, unroll=4)

        def accumulate(buf):
            def acc_row(r, carry):
                for c in range(nc):
                    sl = pl.ds(c * LANES, LANES)
                    plsc.addupdate(acc_v.at[r, sl], buf[r, sl])
                return carry

            lax.fori_loop(0, bw, acc_row, 0, unroll=4)

        pltpu.async_copy(table_hbm.at[idx_v.at[0]], buf0, sem0)

        def step(i, carry):
            s0 = 2 * i
            pltpu.async_copy(table_hbm.at[idx_v.at[s0 + 1]], buf1, sem1)
            pltpu.make_async_copy(table_hbm.at[idx_v.at[s0]], buf0, sem0).wait()
            accumulate(buf0)
            nxt = lax.min(s0 + 2, S - 1)
            pltpu.async_copy(table_hbm.at[idx_v.at[nxt]], buf0, sem0)
            pltpu.make_async_copy(table_hbm.at[idx_v.at[s0 + 1]], buf1, sem1).wait()
            accumulate(buf1)
            return carry

        lax.fori_loop(0, S // 2, step, 0)
        pltpu.make_async_copy(table_hbm.at[idx_v.at[S - 1]], buf0, sem0).wait()

        pltpu.sync_copy(acc_v, out_hbm.at[pl.ds(base, bw)])

    return sc_pool


@functools.cache
def _make_tc_mlp(B, D, H, O):
    def mlp(p_ref, w1_ref, b1_ref, w2_ref, b2_ref, o_ref):
        h = lax.dot_general(
            p_ref[...], w1_ref[...], (((1,), (1,)), ((), ())),
            preferred_element_type=jnp.float32,
        )
        h = jnp.maximum(h + b1_ref[...], 0.0)
        o_ref[...] = lax.dot_general(
            h, w2_ref[...], (((1,), (1,)), ((), ())),
            preferred_element_type=jnp.float32,
        ) + b2_ref[...]

    return pl.pallas_call(
        mlp,
        out_shape=jax.ShapeDtypeStruct((B, O), jnp.float32),
    )


def kernel(x, embed, W1, b1, W2, b2):
    B, S = x.shape
    V, D = embed.shape
    H = W1.shape[0]
    O = W2.shape[0]

    xT = x.T
    pooled_sum = _make_sc_pool(B, S, D, V)(xT, embed)
    W1s = W1 * (1.0 / S)
    out = _make_tc_mlp(B, D, H, O)(
        pooled_sum, W1s, b1.reshape(1, H), W2, b2.reshape(1, O)
    )
    return out


# speedup vs baseline: 16.8783x; 1.7076x over previous
import functools

import jax
import jax.numpy as jnp
from jax import lax
from jax.experimental import pallas as pl
from jax.experimental.pallas import tpu as pltpu
from jax.experimental.pallas import tpu_sc as plsc

NUM_CORES = 2
NUM_SUBCORES = 16
NUM_WORKERS = NUM_CORES * NUM_SUBCORES
LANES = 16
NSLOTS = 4


@functools.cache
def _make_sc_pool(B, S, D, V):
    assert B % NUM_WORKERS == 0
    bw = B // NUM_WORKERS
    assert bw % NSLOTS == 0
    assert D % LANES == 0
    nc = D // LANES
    chunks = []
    off = 0
    while off < S:
        ln = min(128, S - off)
        chunks.append((off, ln))
        off += ln
    assert all(o % 8 == 0 for o, _ in chunks)
    unroll = 4
    assert S % unroll == 0

    mesh = plsc.VectorSubcoreMesh(core_axis_name="c", subcore_axis_name="s")

    @functools.partial(
        pl.kernel,
        mesh=mesh,
        out_type=jax.ShapeDtypeStruct((B, D), jnp.float32),
        scratch_types=[
            pltpu.VMEM((NSLOTS, S), jnp.int32),
            pltpu.VMEM((NSLOTS, S, D), jnp.float32),
            pltpu.VMEM((bw, D), jnp.float32),
        ]
        + [pltpu.SemaphoreType.DMA] * NSLOTS
        + [pltpu.SemaphoreType.DMA] * NSLOTS,
    )
    def sc_pool(x_hbm, table_hbm, out_hbm, idx_v, buf_v, acc_v, *sems):
        sem_i = sems[:NSLOTS]
        sem_g = sems[NSLOTS:]
        wid = lax.axis_index("s") * NUM_CORES + lax.axis_index("c")
        base = wid * bw

        def issue_idx(g, p):
            pltpu.async_copy(x_hbm.at[base + g], idx_v.at[p], sem_i[p])

        def wait_idx(g, p):
            pltpu.make_async_copy(
                x_hbm.at[base + g], idx_v.at[p], sem_i[p]).wait()

        def issue_gathers(p):
            for o, ln in chunks:
                pltpu.async_copy(
                    table_hbm.at[idx_v.at[p, pl.ds(o, ln)]],
                    buf_v.at[p, pl.ds(o, ln)], sem_g[p])

        def wait_gathers(p):
            for o, ln in chunks:
                pltpu.make_async_copy(
                    table_hbm.at[idx_v.at[p, pl.ds(o, ln)]],
                    buf_v.at[p, pl.ds(o, ln)], sem_g[p]).wait()

        for p in range(NSLOTS):
            issue_idx(p, p)
        for p in range(NSLOTS - 1):
            wait_idx(p, p)
            issue_gathers(p)

        zeros = jnp.zeros((LANES,), jnp.float32)

        def step(g, p):
            wait_gathers(p)
            nxt = g + NSLOTS - 1

            @pl.when(nxt < bw)
            def _():
                wait_idx(nxt, (p + NSLOTS - 1) % NSLOTS)
                issue_gathers((p + NSLOTS - 1) % NSLOTS)

            @pl.when(g + NSLOTS < bw)
            def _():
                issue_idx(g + NSLOTS, p)

            def body(s, accs):
                new = list(accs)
                for u in range(unroll):
                    for c in range(nc):
                        new[c] = new[c] + buf_v[
                            p, s * unroll + u, pl.ds(c * LANES, LANES)]
                return tuple(new)

            accs = lax.fori_loop(0, S // unroll, body, (zeros,) * nc,
                                 unroll=1)
            for c in range(nc):
                acc_v[g, pl.ds(c * LANES, LANES)] = accs[c]

        def outer(i, carry):
            for p in range(NSLOTS):
                step(i * NSLOTS + p, p)
            return carry

        lax.fori_loop(0, bw // NSLOTS, outer, 0)

        pltpu.sync_copy(acc_v, out_hbm.at[pl.ds(base, bw)])

    return sc_pool


@functools.cache
def _make_tc_mlp(B, D, H, O):
    def mlp(p_ref, w1_ref, b1_ref, w2_ref, b2_ref, o_ref):
        h = lax.dot_general(
            p_ref[...], w1_ref[...], (((1,), (1,)), ((), ())),
            preferred_element_type=jnp.float32,
        )
        h = jnp.maximum(h + b1_ref[...], 0.0)
        o_ref[...] = lax.dot_general(
            h, w2_ref[...], (((1,), (1,)), ((), ())),
            preferred_element_type=jnp.float32,
        ) + b2_ref[...]

    return pl.pallas_call(
        mlp,
        out_shape=jax.ShapeDtypeStruct((B, O), jnp.float32),
    )


def kernel(x, embed, W1, b1, W2, b2):
    B, S = x.shape
    V, D = embed.shape
    H = W1.shape[0]
    O = W2.shape[0]

    pooled_sum = _make_sc_pool(B, S, D, V)(x, embed)
    W1s = W1 * (1.0 / S)
    out = _make_tc_mlp(B, D, H, O)(
        pooled_sum, W1s, b1.reshape(1, H), W2, b2.reshape(1, O)
    )
    return out
